# baseline (device time: 42260 ns/iter reference)
import jax
import jax.numpy as jnp
from jax import lax
from jax.experimental import pallas as pl
from jax.experimental.pallas import tpu as pltpu

B, SQ, H, D = 16, 1, 16, 64
HD = H * D
BH = B // 2
SCALE = D ** -0.5


def kernel(Q, K, V):
    kv = K.shape[1]

    Kt = jnp.transpose(K, (0, 2, 3, 1)).reshape(B, HD, kv)
    Vt = jnp.transpose(V, (0, 2, 3, 1)).reshape(B, HD, kv)
    Qr = Q.reshape(B, SQ, HD)

    def body(xsel_ref, q_ref, k_hbm, v_hbm, out_ref,
             kbuf, vbuf, oacc, macc, lacc, peer_o, peer_m, peer_l, peer_x,
             copy_sems, send_sems, recv_sems):
        b = pl.program_id(0)
        xs = xsel_ref[0]

        def copies(slot, blk):
            return (
                pltpu.make_async_copy(k_hbm.at[xs * BH + blk],
                                      kbuf.at[slot], copy_sems.at[0, slot]),
                pltpu.make_async_copy(v_hbm.at[xs * BH + blk],
                                      vbuf.at[slot], copy_sems.at[1, slot]),
            )

        @pl.when(b == 0)
        def _():
            for c in copies(0, 0):
                c.start()

        @pl.when(b < BH - 1)
        def _():
            for c in copies((b + 1) % 2, b + 1):
                c.start()

        for c in copies(b % 2, b):
            c.wait()

        q_row = q_ref[0]
        k2 = kbuf[b % 2]
        v2 = vbuf[b % 2]

        eh = lax.broadcasted_iota(jnp.int32, (H, HD), 0)
        ec = lax.broadcasted_iota(jnp.int32, (H, HD), 1)
        qrow = jnp.where(ec // D == eh, q_row, 0.0)

        th = lax.broadcasted_iota(jnp.int32, (HD, H), 0)
        tc = lax.broadcasted_iota(jnp.int32, (HD, H), 1)
        emaskT = (th // D == tc).astype(jnp.float32)

        s = lax.dot_general(
            qrow, k2, (((1,), (0,)), ((), ())),
            preferred_element_type=jnp.float32) * SCALE
        m_b = jnp.max(s, axis=1, keepdims=True)
        p = jnp.exp(s - m_b)
        l_b = jnp.sum(p, axis=1, keepdims=True)

        p_wide = lax.dot_general(
            emaskT, p, (((1,), (0,)), ((), ())),
            preferred_element_type=jnp.float32)
        o_col = jnp.sum(v2 * p_wide, axis=1, keepdims=True)

        sel_o = lax.broadcasted_iota(jnp.int32, (HD, BH), 1) == b
        sel_s = lax.broadcasted_iota(jnp.int32, (H, BH), 1) == b
        oacc[...] = jnp.where(sel_o, o_col, oacc[...])
        macc[...] = jnp.where(sel_s, m_b, macc[...])
        lacc[...] = jnp.where(sel_s, l_b, lacc[...])

        @pl.when(b == BH - 1)
        def _():
            my_x = lax.axis_index("x")
            my_y = lax.axis_index("y")
            x_peer = (1 - my_x, my_y)
            y_peer = (my_x, 1 - my_y)

            barrier = pltpu.get_barrier_semaphore()
            for nbr in (x_peer, y_peer):
                pl.semaphore_signal(barrier, inc=1, device_id=nbr,
                                    device_id_type=pl.DeviceIdType.MESH)
            pl.semaphore_wait(barrier, 2)

            rdmas = []
            for i, (src, dst) in enumerate(
                    ((oacc, peer_o), (macc, peer_m), (lacc, peer_l))):
                rdma = pltpu.make_async_remote_copy(
                    src_ref=src, dst_ref=dst,
                    send_sem=send_sems.at[i], recv_sem=recv_sems.at[i],
                    device_id=y_peer, device_id_type=pl.DeviceIdType.MESH)
                rdma.start()
                rdmas.append(rdma)
            for rdma in rdmas:
                rdma.wait()

            mm = jnp.maximum(macc[...], peer_m[...])
            a_l = jnp.exp(macc[...] - mm)
            a_p = jnp.exp(peer_m[...] - mm)
            l_tot = a_l * lacc[...] + a_p * peer_l[...]

            def widen(x):
                return lax.dot_general(
                    emaskT, x, (((1,), (0,)), ((), ())),
                    preferred_element_type=jnp.float32)

            o_mine = (widen(a_l) * oacc[...] +
                      widen(a_p) * peer_o[...]) / widen(l_tot)
            oacc[...] = o_mine

            rdma_x = pltpu.make_async_remote_copy(
                src_ref=oacc, dst_ref=peer_x,
                send_sem=send_sems.at[3], recv_sem=recv_sems.at[3],
                device_id=x_peer, device_id_type=pl.DeviceIdType.MESH)
            rdma_x.start()
            rdma_x.wait()

            mine2 = jnp.concatenate([o_mine, o_mine], axis=1)
            theirs2 = jnp.concatenate([peer_x[...], peer_x[...]], axis=1)
            col = lax.broadcasted_iota(jnp.int32, (HD, B), 1) // BH
            out_ref[...] = jnp.where(col == my_x, mine2, theirs2)

    res = pl.pallas_call(
        body,
        grid_spec=pltpu.PrefetchScalarGridSpec(
            num_scalar_prefetch=1,
            grid=(BH,),
            in_specs=[
                pl.BlockSpec((1, SQ, HD), lambda b, xsel: (xsel[0] * BH + b, 0, 0)),
                pl.BlockSpec(memory_space=pl.ANY),
                pl.BlockSpec(memory_space=pl.ANY),
            ],
            out_specs=pl.BlockSpec((HD, B), lambda b, xsel: (0, 0)),
            scratch_shapes=[
                pltpu.VMEM((2, HD, kv), jnp.float32),
                pltpu.VMEM((2, HD, kv), jnp.float32),
                pltpu.VMEM((HD, BH), jnp.float32),
                pltpu.VMEM((H, BH), jnp.float32),
                pltpu.VMEM((H, BH), jnp.float32),
                pltpu.VMEM((HD, BH), jnp.float32),
                pltpu.VMEM((H, BH), jnp.float32),
                pltpu.VMEM((H, BH), jnp.float32),
                pltpu.VMEM((HD, BH), jnp.float32),
                pltpu.SemaphoreType.DMA((2, 2)),
                pltpu.SemaphoreType.DMA((4,)),
                pltpu.SemaphoreType.DMA((4,)),
            ],
        ),
        out_shape=jax.ShapeDtypeStruct((HD, B), jnp.float32),
        compiler_params=pltpu.CompilerParams(
            collective_id=0,
            dimension_semantics=("arbitrary",),
        ),
    )(jnp.reshape(lax.axis_index("x"), (1,)).astype(jnp.int32), Qr, Kt, Vt)
    return jnp.transpose(res, (1, 0)).reshape(B, SQ, H, D)


# device time: 31390 ns/iter; 1.3463x vs baseline; 1.3463x over previous
import jax
import jax.numpy as jnp
from jax import lax
from jax.experimental import pallas as pl
from jax.experimental.pallas import tpu as pltpu

B, SQ, H, D = 16, 1, 16, 64
HD = H * D
BH = B // 2
SCALE = D ** -0.5


def kernel(Q, K, V):
    kv = K.shape[1]

    Kt = jnp.transpose(K, (0, 2, 3, 1)).reshape(B, HD, kv)
    Vt = jnp.transpose(V, (0, 2, 3, 1)).reshape(B, HD, kv)
    Qr = Q.reshape(B, SQ, HD)

    def body(xsel_ref, q_ref, k_hbm, v_hbm, out_ref,
             kbuf, vbuf, oacc, macc, lacc, peer_o, peer_m, peer_l, peer_x,
             copy_sems, send_sems, recv_sems):
        b = pl.program_id(0)
        xs = xsel_ref[0]

        def copies(slot, blk):
            return (
                pltpu.make_async_copy(k_hbm.at[xs * BH + blk],
                                      kbuf.at[slot], copy_sems.at[0, slot]),
                pltpu.make_async_copy(v_hbm.at[xs * BH + blk],
                                      vbuf.at[slot], copy_sems.at[1, slot]),
            )

        @pl.when(b == 0)
        def _():
            for c in copies(0, 0):
                c.start()

        @pl.when(b < BH - 1)
        def _():
            for c in copies((b + 1) % 2, b + 1):
                c.start()

        for c in copies(b % 2, b):
            c.wait()

        q_row = q_ref[0]
        k2 = kbuf[b % 2]
        v2 = vbuf[b % 2]

        eh = lax.broadcasted_iota(jnp.int32, (H, HD), 0)
        ec = lax.broadcasted_iota(jnp.int32, (H, HD), 1)
        head_of = ec // D == eh
        qrow = jnp.where(head_of, q_row, 0.0)
        emask = head_of.astype(jnp.float32)

        th = lax.broadcasted_iota(jnp.int32, (HD, H), 0)
        tc = lax.broadcasted_iota(jnp.int32, (HD, H), 1)
        emaskT = (th // D == tc).astype(jnp.float32)

        s = lax.dot_general(
            qrow, k2, (((1,), (0,)), ((), ())),
            preferred_element_type=jnp.float32) * SCALE
        m_b = jnp.max(s, axis=1, keepdims=True)
        p = jnp.exp(s - m_b)
        l_b = jnp.sum(p, axis=1, keepdims=True)

        p_wide = lax.dot_general(
            emaskT, p, (((1,), (0,)), ((), ())),
            preferred_element_type=jnp.float32)
        o_col = jnp.sum(v2 * p_wide, axis=1, keepdims=True)
        o_row = jnp.transpose(o_col)
        m_row = jnp.transpose(m_b)
        l_row = jnp.transpose(l_b)

        sel_o = lax.broadcasted_iota(jnp.int32, (BH, HD), 0) == b
        sel_s = lax.broadcasted_iota(jnp.int32, (BH, H), 0) == b
        oacc[...] = jnp.where(sel_o, o_row, oacc[...])
        macc[...] = jnp.where(sel_s, m_row, macc[...])
        lacc[...] = jnp.where(sel_s, l_row, lacc[...])

        @pl.when(b == BH - 1)
        def _():
            my_x = lax.axis_index("x")
            my_y = lax.axis_index("y")
            x_peer = (1 - my_x, my_y)
            y_peer = (my_x, 1 - my_y)

            barrier = pltpu.get_barrier_semaphore()
            for nbr in (x_peer, y_peer):
                pl.semaphore_signal(barrier, inc=1, device_id=nbr,
                                    device_id_type=pl.DeviceIdType.MESH)
            pl.semaphore_wait(barrier, 2)

            rdmas = []
            for i, (src, dst) in enumerate(
                    ((oacc, peer_o), (macc, peer_m), (lacc, peer_l))):
                rdma = pltpu.make_async_remote_copy(
                    src_ref=src, dst_ref=dst,
                    send_sem=send_sems.at[i], recv_sem=recv_sems.at[i],
                    device_id=y_peer, device_id_type=pl.DeviceIdType.MESH)
                rdma.start()
                rdmas.append(rdma)
            for rdma in rdmas:
                rdma.wait()

            mm = jnp.maximum(macc[...], peer_m[...])
            a_l = jnp.exp(macc[...] - mm)
            a_p = jnp.exp(peer_m[...] - mm)
            l_tot = a_l * lacc[...] + a_p * peer_l[...]

            def widen(x):
                return lax.dot_general(
                    x, emask, (((1,), (0,)), ((), ())),
                    preferred_element_type=jnp.float32)

            o_mine = (widen(a_l) * oacc[...] +
                      widen(a_p) * peer_o[...]) / widen(l_tot)
            oacc[...] = o_mine

            rdma_x = pltpu.make_async_remote_copy(
                src_ref=oacc, dst_ref=peer_x,
                send_sem=send_sems.at[3], recv_sem=recv_sems.at[3],
                device_id=x_peer, device_id_type=pl.DeviceIdType.MESH)
            rdma_x.start()
            rdma_x.wait()

            mine2 = jnp.concatenate([o_mine, o_mine], axis=0)
            theirs2 = jnp.concatenate([peer_x[...], peer_x[...]], axis=0)
            row = lax.broadcasted_iota(jnp.int32, (B, HD), 0) // BH
            out_ref[...] = jnp.where(row == my_x, mine2, theirs2)

    res = pl.pallas_call(
        body,
        grid_spec=pltpu.PrefetchScalarGridSpec(
            num_scalar_prefetch=1,
            grid=(BH,),
            in_specs=[
                pl.BlockSpec((1, SQ, HD), lambda b, xsel: (xsel[0] * BH + b, 0, 0)),
                pl.BlockSpec(memory_space=pl.ANY),
                pl.BlockSpec(memory_space=pl.ANY),
            ],
            out_specs=pl.BlockSpec((B, HD), lambda b, xsel: (0, 0)),
            scratch_shapes=[
                pltpu.VMEM((2, HD, kv), jnp.float32),
                pltpu.VMEM((2, HD, kv), jnp.float32),
                pltpu.VMEM((BH, HD), jnp.float32),
                pltpu.VMEM((BH, H), jnp.float32),
                pltpu.VMEM((BH, H), jnp.float32),
                pltpu.VMEM((BH, HD), jnp.float32),
                pltpu.VMEM((BH, H), jnp.float32),
                pltpu.VMEM((BH, H), jnp.float32),
                pltpu.VMEM((BH, HD), jnp.float32),
                pltpu.SemaphoreType.DMA((2, 2)),
                pltpu.SemaphoreType.DMA((4,)),
                pltpu.SemaphoreType.DMA((4,)),
            ],
        ),
        out_shape=jax.ShapeDtypeStruct((B, HD), jnp.float32),
        compiler_params=pltpu.CompilerParams(
            collective_id=0,
            dimension_semantics=("arbitrary",),
        ),
    )(jnp.reshape(lax.axis_index("x"), (1,)).astype(jnp.int32), Qr, Kt, Vt)
    return res.reshape(B, SQ, H, D)
